# baseline (speedup 1.0000x reference)
import jax
import jax.numpy as jnp
from jax.experimental import pallas as pl

_NUM_NODES_PER_GRAPH = 82
_HID = 30
_ROW_BLOCK = 13056


def _gate_kernel(x_ref, wc_ref, bc_ref, o_ref):
    xb = x_ref[...]
    u = jnp.dot(xb, wc_ref[...], preferred_element_type=jnp.float32) + bc_ref[...]
    a = u[:, :_HID]
    c = u[:, _HID:]
    h = 0.5 * (1.0 - jnp.tanh(a)) * jnp.tanh(c)
    o_ref[...] = jnp.maximum(h, 0.0).astype(jnp.bfloat16)


def _head_kernel(h_ref, w_ref, b_ref, o_ref):
    o_ref[...] = (
        jnp.dot(h_ref[0], w_ref[...], preferred_element_type=jnp.float32)
        + b_ref[...]
    )[None]


@jax.jit
def kernel(x, edge_index, Wxz, bxz, Whz, bhz, Wxr, bxr, Whr, bhr, Wxh, bxh, Whh, bhh, W_lin, b_lin):
    n, d = x.shape
    hid = Wxz.shape[1]
    wc = jnp.concatenate([Wxz * 0.5, Wxh], axis=1)
    bc = jnp.concatenate([(bxz + bhz) * 0.5, bxh + bhh]).reshape(1, 2 * hid)

    grid = pl.cdiv(n, _ROW_BLOCK)
    h = pl.pallas_call(
        _gate_kernel,
        grid=(grid,),
        in_specs=[
            pl.BlockSpec((_ROW_BLOCK, d), lambda i: (i, 0)),
            pl.BlockSpec((d, 2 * hid), lambda i: (0, 0)),
            pl.BlockSpec((1, 2 * hid), lambda i: (0, 0)),
        ],
        out_specs=pl.BlockSpec((_ROW_BLOCK, hid), lambda i: (i, 0)),
        out_shape=jax.ShapeDtypeStruct((n, hid), jnp.bfloat16),
    )(x, wc, bc)

    feat = hid * _NUM_NODES_PER_GRAPH
    g = n // _NUM_NODES_PER_GRAPH
    n_hblk = 1
    gb = g // n_hblk
    hf = h.reshape(n_hblk, gb, feat)
    w2 = W_lin.T.astype(jnp.bfloat16)
    out_dim = w2.shape[1]
    out = pl.pallas_call(
        _head_kernel,
        grid=(n_hblk,),
        in_specs=[
            pl.BlockSpec((1, gb, feat), lambda i: (i, 0, 0)),
            pl.BlockSpec((feat, out_dim), lambda i: (0, 0)),
            pl.BlockSpec((1, out_dim), lambda i: (0, 0)),
        ],
        out_specs=pl.BlockSpec((1, gb, out_dim), lambda i: (i, 0, 0)),
        out_shape=jax.ShapeDtypeStruct((n_hblk, gb, out_dim), jnp.float32),
    )(hf, w2, b_lin.reshape(1, out_dim))
    return out.reshape(g, out_dim)


# speedup vs baseline: 1.0471x; 1.0471x over previous
import jax
import jax.numpy as jnp
from jax.experimental import pallas as pl

_NUM_NODES_PER_GRAPH = 82
_HID = 30
_ROW_BLOCK = 19584


def _gate_kernel(x_ref, wc_ref, bc_ref, o_ref):
    xb = x_ref[...]
    u = jnp.dot(xb, wc_ref[...], preferred_element_type=jnp.float32) + bc_ref[...]
    t = jnp.tanh(u)
    h = (1.0 - t[:, :_HID]) * t[:, _HID:]
    o_ref[...] = jnp.maximum(h, 0.0).astype(jnp.bfloat16)


def _head_kernel(h_ref, w_ref, b_ref, o_ref):
    o_ref[...] = (
        jnp.dot(h_ref[0], w_ref[...], preferred_element_type=jnp.float32)
        + b_ref[...]
    )[None]


@jax.jit
def kernel(x, edge_index, Wxz, bxz, Whz, bhz, Wxr, bxr, Whr, bhr, Wxh, bxh, Whh, bhh, W_lin, b_lin):
    n, d = x.shape
    hid = Wxz.shape[1]
    wc = jnp.concatenate([Wxz * 0.5, Wxh], axis=1)
    bc = jnp.concatenate([(bxz + bhz) * 0.5, bxh + bhh]).reshape(1, 2 * hid)

    grid = pl.cdiv(n, _ROW_BLOCK)
    h = pl.pallas_call(
        _gate_kernel,
        grid=(grid,),
        in_specs=[
            pl.BlockSpec((_ROW_BLOCK, d), lambda i: (i, 0)),
            pl.BlockSpec((d, 2 * hid), lambda i: (0, 0)),
            pl.BlockSpec((1, 2 * hid), lambda i: (0, 0)),
        ],
        out_specs=pl.BlockSpec((_ROW_BLOCK, hid), lambda i: (i, 0)),
        out_shape=jax.ShapeDtypeStruct((n, hid), jnp.bfloat16),
    )(x, wc, bc)

    feat = hid * _NUM_NODES_PER_GRAPH
    g = n // _NUM_NODES_PER_GRAPH
    n_hblk = 1
    gb = g // n_hblk
    hf = h.reshape(n_hblk, gb, feat)
    w2 = (W_lin.T * 0.5).astype(jnp.bfloat16)
    out_dim = w2.shape[1]
    out = pl.pallas_call(
        _head_kernel,
        grid=(n_hblk,),
        in_specs=[
            pl.BlockSpec((1, gb, feat), lambda i: (i, 0, 0)),
            pl.BlockSpec((feat, out_dim), lambda i: (0, 0)),
            pl.BlockSpec((1, out_dim), lambda i: (0, 0)),
        ],
        out_specs=pl.BlockSpec((1, gb, out_dim), lambda i: (i, 0, 0)),
        out_shape=jax.ShapeDtypeStruct((n_hblk, gb, out_dim), jnp.float32),
    )(hf, w2, b_lin.reshape(1, out_dim))
    return out.reshape(g, out_dim)


# speedup vs baseline: 1.0578x; 1.0102x over previous
import jax
import jax.numpy as jnp
from jax.experimental import pallas as pl

_NUM_NODES_PER_GRAPH = 82
_HID = 30
_ROW_BLOCK = 19584


def _gate_kernel(x_ref, wa_ref, wh_ref, ba_ref, bh_ref, o_ref):
    xb = x_ref[...]
    a = jnp.dot(xb, wa_ref[...], preferred_element_type=jnp.float32) + ba_ref[...]
    c = jnp.dot(xb, wh_ref[...], preferred_element_type=jnp.float32) + bh_ref[...]
    h = (1.0 - jnp.tanh(a)) * jnp.tanh(c)
    o_ref[...] = jnp.maximum(h, 0.0).astype(jnp.bfloat16)


def _head_kernel(h_ref, w_ref, b_ref, o_ref):
    o_ref[...] = (
        jnp.dot(h_ref[0], w_ref[...], preferred_element_type=jnp.float32)
        + b_ref[...]
    )[None]


@jax.jit
def kernel(x, edge_index, Wxz, bxz, Whz, bhz, Wxr, bxr, Whr, bhr, Wxh, bxh, Whh, bhh, W_lin, b_lin):
    n, d = x.shape
    hid = Wxz.shape[1]
    wa = Wxz * 0.5
    ba = ((bxz + bhz) * 0.5).reshape(1, hid)
    bh2 = (bxh + bhh).reshape(1, hid)

    grid = pl.cdiv(n, _ROW_BLOCK)
    h = pl.pallas_call(
        _gate_kernel,
        grid=(grid,),
        in_specs=[
            pl.BlockSpec((_ROW_BLOCK, d), lambda i: (i, 0)),
            pl.BlockSpec((d, hid), lambda i: (0, 0)),
            pl.BlockSpec((d, hid), lambda i: (0, 0)),
            pl.BlockSpec((1, hid), lambda i: (0, 0)),
            pl.BlockSpec((1, hid), lambda i: (0, 0)),
        ],
        out_specs=pl.BlockSpec((_ROW_BLOCK, hid), lambda i: (i, 0)),
        out_shape=jax.ShapeDtypeStruct((n, hid), jnp.bfloat16),
    )(x, wa, Wxh, ba, bh2)

    feat = hid * _NUM_NODES_PER_GRAPH
    g = n // _NUM_NODES_PER_GRAPH
    n_hblk = 1
    gb = g // n_hblk
    hf = h.reshape(n_hblk, gb, feat)
    w2 = (W_lin.T * 0.5).astype(jnp.bfloat16)
    out_dim = w2.shape[1]
    out = pl.pallas_call(
        _head_kernel,
        grid=(n_hblk,),
        in_specs=[
            pl.BlockSpec((1, gb, feat), lambda i: (i, 0, 0)),
            pl.BlockSpec((feat, out_dim), lambda i: (0, 0)),
            pl.BlockSpec((1, out_dim), lambda i: (0, 0)),
        ],
        out_specs=pl.BlockSpec((1, gb, out_dim), lambda i: (i, 0, 0)),
        out_shape=jax.ShapeDtypeStruct((n_hblk, gb, out_dim), jnp.float32),
    )(hf, w2, b_lin.reshape(1, out_dim))
    return out.reshape(g, out_dim)
